# baseline (speedup 1.0000x reference)
import functools
import math

import jax
import jax.numpy as jnp
from jax import lax
from jax.experimental import pallas as pl
from jax.experimental.pallas import tpu as pltpu
from jax.experimental.pallas import tpu_sc as plsc

_K = 8
_N = 32
_LANES = 16
_NC = 2
_NS = 16
_NW = _NC * _NS
_GROUPS_PER_W = 196
_G = _NW * _GROUPS_PER_W
_RPAD = _G * _LANES
_NPAD = _G * 8
_BLK = 8 * _N * 2


def _sc_body(scores_hbm, u_hbm, marg_hbm, samp_hbm,
             scv0, scv1, uv0, uv1, wv0, wv1, bt0, bt1, mb0, mb1, sb0, sb1,
             sis0, sis1, siu0, siu1, som0, som1, sos0, sos1):
    wid = lax.axis_index("s") * _NC + lax.axis_index("c")
    lane = lax.iota(jnp.int32, _LANES)
    bidx = lax.shift_right_logical(lane, 1) * (2 * _N) + (lane & 1)
    ones = jnp.full((_LANES,), 1.0, jnp.float32)
    zero = jnp.zeros((_LANES,), jnp.float32)
    base = wid * _GROUPS_PER_W

    scv = (scv0, scv1)
    uv = (uv0, uv1)
    wv = (wv0, wv1)
    bt = (bt0, bt1)
    mb = (mb0, mb1)
    sb = (sb0, sb1)
    sis = (sis0, sis1)
    siu = (siu0, siu1)
    som = (som0, som1)
    sos = (sos0, sos1)

    for btab in bt:
        for i in range(_N + 1):
            btab[i * (_K + 1)] = ones
            for j in range(min(_K, _N - i) + 1, _K + 1):
                btab[i * (_K + 1) + j] = zero

    def start_in(b, g):
        pltpu.async_copy(scores_hbm.at[g], scv[b], sis[b])
        pltpu.async_copy(u_hbm.at[g], uv[b], siu[b])

    start_in(0, base)
    start_in(1, base + 1)

    def compute(b, g):
        theta_v, u_v, w_v, btab, mbuf, sbuf = scv[b], uv[b], wv[b], bt[b], mb[b], sb[b]

        for i in range(_N):
            th = plsc.load_gather(theta_v, [bidx + 2 * i])
            w_v[i] = jnp.exp(th)

        b_st = [ones] + [zero] * _K
        for i in range(_N - 1, -1, -1):
            wi = w_v[i]
            hi = min(_K, _N - i)
            for k in range(hi, 0, -1):
                b_st[k] = b_st[k] + b_st[k - 1] * wi
            for j in range(1, hi + 1):
                btab[i * (_K + 1) + j] = b_st[j]

        f = [ones] + [zero] * _K
        for i in range(_N):
            wi = w_v[i]
            jlo = max(0, i - (_N - _K))
            jhi = min(i, _K - 1)
            num = f[jlo] * btab[(i + 1) * (_K + 1) + (_K - 1 - jlo)]
            for j in range(jlo + 1, jhi + 1):
                num = num + f[j] * btab[(i + 1) * (_K + 1) + (_K - 1 - j)]
            plsc.store_scatter(mbuf, [bidx + 2 * i], wi * num)
            hi = min(_K, i + 1)
            for k in range(hi, 0, -1):
                f[k] = f[k] + f[k - 1] * wi
        inv = 1.0 / f[_K]
        for j in range(_BLK // _LANES):
            mbuf[pl.ds(j * _LANES, _LANES)] = mbuf[pl.ds(j * _LANES, _LANES)] * inv

        r = jnp.full((_LANES,), _K, jnp.int32)
        for i in range(_N):
            rm1 = jnp.maximum(r - 1, 0)
            g1 = plsc.load_gather(btab, [(i + 1) * (_K + 1) + rm1, lane])
            g2 = plsc.load_gather(btab, [i * (_K + 1) + r, lane])
            wi = w_v[i]
            ui = u_v[i]
            take_main = ui * g2 < wi * g1
            take_edge = ui < jnp.minimum(wi, 1.0)
            take = jnp.where(g2 == 0.0, take_edge, take_main) & (r > 0)
            plsc.store_scatter(sbuf, [bidx + 2 * i],
                               jnp.where(take, 1.0, 0.0))
            r = r - jnp.where(take, 1, 0).astype(jnp.int32)

    def super_step(si, _):
        for b in range(2):
            g = base + 2 * si + b
            pltpu.make_async_copy(scores_hbm.at[g], scv[b], sis[b]).wait()
            pltpu.make_async_copy(u_hbm.at[g], uv[b], siu[b]).wait()

            @pl.when(si > 0)
            def _drain_out():
                pltpu.make_async_copy(mb[b], marg_hbm.at[g], som[b]).wait()
                pltpu.make_async_copy(sb[b], samp_hbm.at[g], sos[b]).wait()

            compute(b, g)

            pltpu.async_copy(mb[b], marg_hbm.at[g], som[b])
            pltpu.async_copy(sb[b], samp_hbm.at[g], sos[b])

            @pl.when(si < _GROUPS_PER_W // 2 - 1)
            def _prefetch():
                start_in(b, g + 2)
        return ()

    lax.fori_loop(0, _GROUPS_PER_W // 2, super_step, (), unroll=False)

    for b in range(2):
        g = base + _GROUPS_PER_W - 2 + b
        pltpu.make_async_copy(mb[b], marg_hbm.at[g], som[b]).wait()
        pltpu.make_async_copy(sb[b], samp_hbm.at[g], sos[b]).wait()


@jax.jit
def kernel(scores):
    nnodes, choices, ensemble = scores.shape
    assert choices == _N and 2 ** int(math.log2(choices)) == choices
    rows = nnodes * ensemble

    u = jax.random.uniform(jax.random.key(42), (_N, 1, rows), dtype=scores.dtype)
    u_p = jnp.pad(u[:, 0, :], ((0, 0), (0, _RPAD - rows)))
    u_b = u_p.reshape(_N, _G, _LANES).transpose(1, 0, 2)

    scores_p = jnp.pad(scores, ((0, _NPAD - nnodes), (0, 0), (0, 0)))
    scores_b = scores_p.reshape(_G, _BLK)

    mesh = plsc.VectorSubcoreMesh(core_axis_name="c", subcore_axis_name="s",
                                  num_cores=_NC, num_subcores=_NS)
    marg_b, samp_b = pl.kernel(
        _sc_body,
        out_type=[
            jax.ShapeDtypeStruct((_G, _BLK), jnp.float32),
            jax.ShapeDtypeStruct((_G, _BLK), jnp.float32),
        ],
        mesh=mesh,
        compiler_params=pltpu.CompilerParams(needs_layout_passes=False),
        scratch_types=(
            [pltpu.VMEM((_BLK,), jnp.float32) for _ in range(2)]
            + [pltpu.VMEM((_N, _LANES), jnp.float32) for _ in range(2)]
            + [pltpu.VMEM((_N, _LANES), jnp.float32) for _ in range(2)]
            + [pltpu.VMEM(((_N + 1) * (_K + 1), _LANES), jnp.float32)
               for _ in range(2)]
            + [pltpu.VMEM((_BLK,), jnp.float32) for _ in range(2)]
            + [pltpu.VMEM((_BLK,), jnp.float32) for _ in range(2)]
            + [pltpu.SemaphoreType.DMA for _ in range(8)]
        ),
    )(scores_b, u_b)

    marginals = marg_b.reshape(_NPAD, _N, ensemble)[:nnodes]
    samples = samp_b.reshape(_NPAD, _N, ensemble)[:nnodes][None]
    return samples, marginals


# speedup vs baseline: 7.1686x; 7.1686x over previous
import functools
import math

import jax
import jax.numpy as jnp
from jax import lax
from jax.experimental import pallas as pl
from jax.experimental.pallas import tpu as pltpu
from jax.experimental.pallas import tpu_sc as plsc

_K = 8
_N = 32
_LANES = 16
_NC = 2
_NS = 16
_NW = _NC * _NS
_GROUPS_PER_W = 196
_G = _NW * _GROUPS_PER_W
_RPAD = _G * _LANES


def _sc_body(theta_hbm, u_hbm, marg_hbm, samp_hbm,
             th0, th1, uv0, uv1, wv0, wv1, bt0, bt1, mv0, mv1, sv0, sv1,
             sin0, sin1, sout0, sout1):
    wid = lax.axis_index("s") * _NC + lax.axis_index("c")
    lane = lax.iota(jnp.int32, _LANES)
    ones = jnp.full((_LANES,), 1.0, jnp.float32)
    zero = jnp.zeros((_LANES,), jnp.float32)
    base = wid * _GROUPS_PER_W

    th = (th0, th1)
    uv = (uv0, uv1)
    wv = (wv0, wv1)
    bt = (bt0, bt1)
    mv = (mv0, mv1)
    sv = (sv0, sv1)
    sin = (sin0, sin1)
    sout = (sout0, sout1)

    for btab in bt:
        for i in range(_N + 1):
            btab[i * (_K + 1)] = ones
            for j in range(min(_K, _N - i) + 1, _K + 1):
                btab[i * (_K + 1) + j] = zero

    def start_in(b, g):
        pltpu.async_copy(theta_hbm.at[g], th[b], sin[b])
        pltpu.async_copy(u_hbm.at[g], uv[b], sin[b])

    def wait_in(b, g):
        pltpu.make_async_copy(theta_hbm.at[g], th[b], sin[b]).wait()
        pltpu.make_async_copy(u_hbm.at[g], uv[b], sin[b]).wait()

    def start_out(b, g):
        pltpu.async_copy(mv[b], marg_hbm.at[g], sout[b])
        pltpu.async_copy(sv[b], samp_hbm.at[g], sout[b])

    def wait_out(b, g):
        pltpu.make_async_copy(mv[b], marg_hbm.at[g], sout[b]).wait()
        pltpu.make_async_copy(sv[b], samp_hbm.at[g], sout[b]).wait()

    start_in(0, base)
    start_in(1, base + 1)

    def compute(b):
        theta_v, u_v, w_v, btab, marg_v, samp_v = \
            th[b], uv[b], wv[b], bt[b], mv[b], sv[b]

        for i in range(_N):
            w_v[i] = jnp.exp(theta_v[i])

        b = [ones] + [zero] * _K
        for i in range(_N - 1, -1, -1):
            wi = w_v[i]
            hi = min(_K, _N - i)
            for k in range(hi, 0, -1):
                b[k] = b[k] + b[k - 1] * wi
            for j in range(1, hi + 1):
                btab[i * (_K + 1) + j] = b[j]

        f = [ones] + [zero] * _K
        for i in range(_N):
            wi = w_v[i]
            jlo = max(0, i - (_N - _K))
            jhi = min(i, _K - 1)
            num = f[jlo] * btab[(i + 1) * (_K + 1) + (_K - 1 - jlo)]
            for j in range(jlo + 1, jhi + 1):
                num = num + f[j] * btab[(i + 1) * (_K + 1) + (_K - 1 - j)]
            marg_v[i] = wi * num
            hi = min(_K, i + 1)
            for k in range(hi, 0, -1):
                f[k] = f[k] + f[k - 1] * wi
        inv = 1.0 / f[_K]
        for i in range(_N):
            marg_v[i] = marg_v[i] * inv

        r = jnp.full((_LANES,), _K, jnp.int32)
        for i in range(_N):
            rm1 = jnp.maximum(r - 1, 0)
            g1 = plsc.load_gather(btab, [(i + 1) * (_K + 1) + rm1, lane])
            g2 = plsc.load_gather(btab, [i * (_K + 1) + r, lane])
            wi = w_v[i]
            ui = u_v[i]
            take_main = ui * g2 < wi * g1
            take_edge = ui < jnp.minimum(wi, 1.0)
            take = jnp.where(g2 == 0.0, take_edge, take_main) & (r > 0)
            samp_v[i] = jnp.where(take, 1.0, 0.0)
            r = r - jnp.where(take, 1, 0).astype(jnp.int32)

    def super_step(si, _):
        for b in range(2):
            g = base + 2 * si + b
            wait_in(b, g)

            @pl.when(si > 0)
            def _drain_out():
                wait_out(b, g)

            compute(b)
            start_out(b, g)

            @pl.when(si < _GROUPS_PER_W // 2 - 1)
            def _prefetch():
                start_in(b, g + 2)
        return ()

    lax.fori_loop(0, _GROUPS_PER_W // 2, super_step, (), unroll=False)

    for b in range(2):
        wait_out(b, base + _GROUPS_PER_W - 2 + b)


@jax.jit
def kernel(scores):
    nnodes, choices, ensemble = scores.shape
    assert choices == _N and 2 ** int(math.log2(choices)) == choices
    rows = nnodes * ensemble
    theta = jnp.transpose(scores, (0, 2, 1)).reshape(rows, choices)

    u = jax.random.uniform(jax.random.key(42), (_N, 1, rows), dtype=theta.dtype)
    u2 = u[:, 0, :]

    pad = _RPAD - rows
    theta_p = jnp.pad(theta, ((0, pad), (0, 0)))
    u_p = jnp.pad(u2, ((0, 0), (0, pad)), constant_values=0.5)
    theta_b = theta_p.reshape(_G, _LANES, _N).transpose(0, 2, 1)
    u_b = u_p.reshape(_N, _G, _LANES).transpose(1, 0, 2)

    mesh = plsc.VectorSubcoreMesh(core_axis_name="c", subcore_axis_name="s",
                                  num_cores=_NC, num_subcores=_NS)
    marg_b, samp_b = pl.kernel(
        _sc_body,
        out_type=[
            jax.ShapeDtypeStruct((_G, _N, _LANES), jnp.float32),
            jax.ShapeDtypeStruct((_G, _N, _LANES), jnp.float32),
        ],
        mesh=mesh,
        compiler_params=pltpu.CompilerParams(needs_layout_passes=False),
        scratch_types=(
            [pltpu.VMEM((_N, _LANES), jnp.float32) for _ in range(2)]
            + [pltpu.VMEM((_N, _LANES), jnp.float32) for _ in range(2)]
            + [pltpu.VMEM((_N, _LANES), jnp.float32) for _ in range(2)]
            + [pltpu.VMEM(((_N + 1) * (_K + 1), _LANES), jnp.float32)
               for _ in range(2)]
            + [pltpu.VMEM((_N, _LANES), jnp.float32) for _ in range(2)]
            + [pltpu.VMEM((_N, _LANES), jnp.float32) for _ in range(2)]
            + [pltpu.SemaphoreType.DMA for _ in range(4)]
        ),
    )(theta_b, u_b)

    marg_flat = marg_b.transpose(0, 2, 1).reshape(_RPAD, _N)[:rows]
    samp_flat = samp_b.transpose(0, 2, 1).reshape(_RPAD, _N)[:rows]
    marginals = jnp.transpose(marg_flat.reshape(nnodes, ensemble, choices), (0, 2, 1))
    samples = jnp.transpose(samp_flat.reshape(nnodes, ensemble, choices), (0, 2, 1))[None]
    return samples, marginals
